# baseline (speedup 1.0000x reference)
import jax
import jax.numpy as jnp
from jax.experimental import pallas as pl

_BB = 4


def _vq_body(x_ref, cb_ref, out_ref):
    bb, t, d = x_ref.shape
    x = x_ref[...].reshape(bb * t, d)
    cb = cb_ref[...]
    st = jax.lax.dot_general(
        cb, x, (((1,), (1,)), ((), ())),
        preferred_element_type=jnp.float32,
    )
    hc = 0.5 * jnp.sum(cb * cb, axis=1, keepdims=True)
    h = st - hc
    out_ref[...] = jnp.argmax(h, axis=0).astype(jnp.int32).reshape(1, 1, -1)


def kernel(z, codebook):
    B, T, D = z.shape
    K = codebook.shape[0]
    G = B // _BB

    R = _BB * T
    out = pl.pallas_call(
        _vq_body,
        grid=(G,),
        in_specs=[
            pl.BlockSpec((_BB, T, D), lambda i: (i, 0, 0)),
            pl.BlockSpec((K, D), lambda i: (0, 0)),
        ],
        out_specs=pl.BlockSpec((1, 1, R), lambda i: (i, 0, 0)),
        out_shape=jax.ShapeDtypeStruct((G, 1, R), jnp.int32),
    )(z, codebook)
    return out.reshape(B, T)


# speedup vs baseline: 1.0321x; 1.0321x over previous
import jax
import jax.numpy as jnp
from jax.experimental import pallas as pl

_BB = 8


def _vq_body(x_ref, cb_ref, out_ref):
    bb, t, d = x_ref.shape
    x = x_ref[...].reshape(bb * t, d)
    cb = cb_ref[...]
    st = jax.lax.dot_general(
        cb, x, (((1,), (1,)), ((), ())),
        preferred_element_type=jnp.float32,
    )
    hc = 0.5 * jnp.sum(cb * cb, axis=1, keepdims=True)
    h = st - hc
    out_ref[...] = jnp.argmax(h, axis=0).astype(jnp.int32).reshape(1, 1, -1)


def kernel(z, codebook):
    B, T, D = z.shape
    K = codebook.shape[0]
    G = B // _BB

    R = _BB * T
    out = pl.pallas_call(
        _vq_body,
        grid=(G,),
        in_specs=[
            pl.BlockSpec((_BB, T, D), lambda i: (i, 0, 0)),
            pl.BlockSpec((K, D), lambda i: (0, 0)),
        ],
        out_specs=pl.BlockSpec((1, 1, R), lambda i: (i, 0, 0)),
        out_shape=jax.ShapeDtypeStruct((G, 1, R), jnp.int32),
    )(z, codebook)
    return out.reshape(B, T)
